# initial kernel scaffold (unmeasured)
import jax
import jax.numpy as jnp
from jax import lax
from jax.experimental import pallas as pl
from jax.experimental.pallas import tpu as pltpu


def kernel(
    x,
):
    def body(*refs):
        pass

    out_shape = jax.ShapeDtypeStruct(..., jnp.float32)
    return pl.pallas_call(body, out_shape=out_shape)(...)


# baseline (device time: 210208 ns/iter reference)
import jax
import jax.numpy as jnp
from jax import lax
from jax.experimental import pallas as pl
from jax.experimental.pallas import tpu as pltpu

M = 8192
N = 1024
CH = 8
R = M // CH


def kernel(x):
    def body(x_ref, out_ref, sx, rx, send_sems, recv_sems):
        i = pl.program_id(0)
        my_x = lax.axis_index("x")
        my_y = lax.axis_index("y")

        @pl.when(i == 0)
        def _():
            barrier = pltpu.get_barrier_semaphore()
            pl.semaphore_signal(
                barrier, inc=1,
                device_id=(1 - my_x, my_y),
                device_id_type=pl.DeviceIdType.MESH,
            )
            pl.semaphore_wait(barrier, 1)

        def step(slot):
            @pl.when(my_x == 0)
            def _():
                sx[slot] = x_ref[0, :, N:].astype(jnp.bfloat16)

            @pl.when(my_x == 1)
            def _():
                sx[slot] = x_ref[0, :, :N].astype(jnp.bfloat16)

            rdma = pltpu.make_async_remote_copy(
                src_ref=sx.at[slot],
                dst_ref=rx.at[slot],
                send_sem=send_sems.at[slot],
                recv_sem=recv_sems.at[slot],
                device_id=(1 - my_x, my_y),
                device_id_type=pl.DeviceIdType.MESH,
            )
            rdma.start()
            rdma.wait()

            @pl.when(my_x == 0)
            def _():
                out_ref[...] = (
                    x_ref[0, :, :N] + rx[slot].astype(jnp.float32)
                ).astype(jnp.bfloat16)

            @pl.when(my_x == 1)
            def _():
                out_ref[...] = (
                    x_ref[0, :, N:] + rx[slot].astype(jnp.float32)
                ).astype(jnp.bfloat16)

        slot_t = lax.rem(i, 2)

        @pl.when(slot_t == 0)
        def _():
            step(0)

        @pl.when(slot_t == 1)
        def _():
            step(1)

    return pl.pallas_call(
        body,
        grid=(CH,),
        in_specs=[pl.BlockSpec((1, R, 2 * N), lambda i: (0, i, 0))],
        out_specs=pl.BlockSpec((R, N), lambda i: (i, 0)),
        out_shape=jax.ShapeDtypeStruct((M, N), jnp.bfloat16),
        scratch_shapes=[
            pltpu.VMEM((2, R, N), jnp.bfloat16),
            pltpu.VMEM((2, R, N), jnp.bfloat16),
            pltpu.SemaphoreType.DMA((2,)),
            pltpu.SemaphoreType.DMA((2,)),
        ],
        compiler_params=pltpu.CompilerParams(
            collective_id=0,
            dimension_semantics=("arbitrary",),
        ),
    )(x)


# device time: 139480 ns/iter; 1.5071x vs baseline; 1.5071x over previous
import jax
import jax.numpy as jnp
from jax import lax
from jax.experimental import pallas as pl
from jax.experimental.pallas import tpu as pltpu

M = 8192
N = 1024
HALF = M // 2
C = 16
R = HALF // C


def kernel(x):
    def body(
        x_hbm, out_hbm,
        ls, la, lf, sx, rx, rf, obd, obf,
        ls_sem, la_sem, lf_sem, od_sem, of_sem,
        xs_sem, xr_sem, fs_sem, fr_sem,
    ):
        my_x = lax.axis_index("x")
        my_y = lax.axis_index("y")
        ox = my_x * N
        oxo = (1 - my_x) * N
        rbd = my_y * HALF
        rbf = (1 - my_y) * HALF

        barrier = pltpu.get_barrier_semaphore()
        pl.semaphore_signal(
            barrier, inc=1, device_id=(1 - my_x, my_y),
            device_id_type=pl.DeviceIdType.MESH,
        )
        pl.semaphore_signal(
            barrier, inc=1, device_id=(my_x, 1 - my_y),
            device_id_type=pl.DeviceIdType.MESH,
        )
        pl.semaphore_wait(barrier, 2)

        def ls_copy(c, s):
            return pltpu.make_async_copy(
                x_hbm.at[0, pl.ds(rbd + c * R, R), pl.ds(oxo, N)],
                ls.at[s], ls_sem.at[s],
            )

        def la_copy(c, s):
            return pltpu.make_async_copy(
                x_hbm.at[0, pl.ds(rbd + c * R, R), pl.ds(ox, N)],
                la.at[s], la_sem.at[s],
            )

        def lf_copy(c, s):
            return pltpu.make_async_copy(
                x_hbm.at[0, pl.ds(rbf + c * R, R), pl.ds(ox, N)],
                lf.at[s], lf_sem.at[s],
            )

        def x_rdma(c):
            return pltpu.make_async_remote_copy(
                src_ref=sx.at[c], dst_ref=rx.at[c],
                send_sem=xs_sem.at[c], recv_sem=xr_sem.at[c],
                device_id=(1 - my_x, my_y),
                device_id_type=pl.DeviceIdType.MESH,
            )

        def f_rdma(c):
            return pltpu.make_async_remote_copy(
                src_ref=rx.at[c], dst_ref=rf.at[c],
                send_sem=fs_sem.at[c], recv_sem=fr_sem.at[c],
                device_id=(my_x, 1 - my_y),
                device_id_type=pl.DeviceIdType.MESH,
            )

        def od_copy(c, s):
            return pltpu.make_async_copy(
                obd.at[s], out_hbm.at[pl.ds(rbd + c * R, R), :], od_sem.at[s]
            )

        def of_copy(c, s):
            return pltpu.make_async_copy(
                obf.at[s], out_hbm.at[pl.ds(rbf + c * R, R), :], of_sem.at[s]
            )

        ls_copy(0, 0).start()
        ls_copy(1, 1).start()
        for c in range(C):
            s = c % 2
            ls_copy(c, s).wait()
            sx[c] = ls[s].astype(jnp.bfloat16)
            x_rdma(c).start()
            if c + 2 < C:
                ls_copy(c + 2, s).start()

        la_copy(0, 0).start()
        la_copy(1, 1).start()
        lf_copy(0, 0).start()
        lf_copy(1, 1).start()
        for c in range(C):
            s = c % 2
            x_rdma(c).wait_recv()
            f_rdma(c).start()

            la_copy(c, s).wait()
            if c >= 2:
                od_copy(c - 2, s).wait()
            obd[s] = (la[s] + rx[c].astype(jnp.float32)).astype(jnp.bfloat16)
            od_copy(c, s).start()
            if c + 2 < C:
                la_copy(c + 2, s).start()

            f_rdma(c).wait_recv()
            lf_copy(c, s).wait()
            if c >= 2:
                of_copy(c - 2, s).wait()
            obf[s] = (lf[s] + rf[c].astype(jnp.float32)).astype(jnp.bfloat16)
            of_copy(c, s).start()
            if c + 2 < C:
                lf_copy(c + 2, s).start()

        for c in (C - 2, C - 1):
            od_copy(c, c % 2).wait()
            of_copy(c, c % 2).wait()
        for c in range(C):
            x_rdma(c).wait_send()
            f_rdma(c).wait_send()

    return pl.pallas_call(
        body,
        in_specs=[pl.BlockSpec(memory_space=pl.MemorySpace.ANY)],
        out_specs=pl.BlockSpec(memory_space=pl.MemorySpace.ANY),
        out_shape=jax.ShapeDtypeStruct((M, N), jnp.bfloat16),
        scratch_shapes=[
            pltpu.VMEM((2, R, N), jnp.float32),
            pltpu.VMEM((2, R, N), jnp.float32),
            pltpu.VMEM((2, R, N), jnp.float32),
            pltpu.VMEM((C, R, N), jnp.bfloat16),
            pltpu.VMEM((C, R, N), jnp.bfloat16),
            pltpu.VMEM((C, R, N), jnp.bfloat16),
            pltpu.VMEM((2, R, N), jnp.bfloat16),
            pltpu.VMEM((2, R, N), jnp.bfloat16),
            pltpu.SemaphoreType.DMA((2,)),
            pltpu.SemaphoreType.DMA((2,)),
            pltpu.SemaphoreType.DMA((2,)),
            pltpu.SemaphoreType.DMA((2,)),
            pltpu.SemaphoreType.DMA((2,)),
            pltpu.SemaphoreType.DMA((C,)),
            pltpu.SemaphoreType.DMA((C,)),
            pltpu.SemaphoreType.DMA((C,)),
            pltpu.SemaphoreType.DMA((C,)),
        ],
        compiler_params=pltpu.CompilerParams(collective_id=0),
    )(x)


# device time: 112618 ns/iter; 1.8666x vs baseline; 1.2385x over previous
import jax
import jax.numpy as jnp
from jax import lax
from jax.experimental import pallas as pl
from jax.experimental.pallas import tpu as pltpu

M = 8192
N = 1024
HALF = M // 2
C = 16
R = HALF // C


def kernel(x):
    def body(
        x_hbm, out_hbm,
        ls, la, lf, sx, rx, rf, obd, obf,
        ls_sem, la_sem, lf_sem, od_sem, of_sem,
        xs_sem, xr_sem, fs_sem, fr_sem,
    ):
        my_x = lax.axis_index("x")
        my_y = lax.axis_index("y")
        ox = my_x * N
        oxo = (1 - my_x) * N
        rbd = my_y * HALF
        rbf = (1 - my_y) * HALF

        barrier = pltpu.get_barrier_semaphore()
        pl.semaphore_signal(
            barrier, inc=1, device_id=(1 - my_x, my_y),
            device_id_type=pl.DeviceIdType.MESH,
        )
        pl.semaphore_signal(
            barrier, inc=1, device_id=(my_x, 1 - my_y),
            device_id_type=pl.DeviceIdType.MESH,
        )
        pl.semaphore_wait(barrier, 2)

        def ls_copy(c, s):
            return pltpu.make_async_copy(
                x_hbm.at[0, pl.ds(rbd + c * R, R), pl.ds(oxo, N)],
                ls.at[s], ls_sem.at[s],
            )

        def la_copy(c, s):
            return pltpu.make_async_copy(
                x_hbm.at[0, pl.ds(rbd + c * R, R), pl.ds(ox, N)],
                la.at[s], la_sem.at[s],
            )

        def lf_copy(c, s):
            return pltpu.make_async_copy(
                x_hbm.at[0, pl.ds(rbf + c * R, R), pl.ds(ox, N)],
                lf.at[s], lf_sem.at[s],
            )

        def x_rdma(c):
            return pltpu.make_async_remote_copy(
                src_ref=sx.at[c], dst_ref=rx.at[c],
                send_sem=xs_sem.at[c], recv_sem=xr_sem.at[c],
                device_id=(1 - my_x, my_y),
                device_id_type=pl.DeviceIdType.MESH,
            )

        def f_rdma(c):
            return pltpu.make_async_remote_copy(
                src_ref=rx.at[c], dst_ref=rf.at[c],
                send_sem=fs_sem.at[c], recv_sem=fr_sem.at[c],
                device_id=(my_x, 1 - my_y),
                device_id_type=pl.DeviceIdType.MESH,
            )

        def od_copy(c, s):
            return pltpu.make_async_copy(
                obd.at[s], out_hbm.at[pl.ds(rbd + c * R, R), :], od_sem.at[s]
            )

        def of_copy(c, s):
            return pltpu.make_async_copy(
                obf.at[s], out_hbm.at[pl.ds(rbf + c * R, R), :], of_sem.at[s]
            )

        ls_copy(0, 0).start()
        ls_copy(1, 1).start()
        for c in range(C):
            s = c % 2
            ls_copy(c, s).wait()
            sx[c] = ls[s].astype(jnp.bfloat16)
            x_rdma(c).start()
            if c + 2 < C:
                ls_copy(c + 2, s).start()

        L = 2
        la_copy(0, 0).start()
        la_copy(1, 1).start()
        lf_copy(0, 0).start()
        lf_copy(1, 1).start()
        for c in range(C + L):
            if c < C:
                s = c % 2
                x_rdma(c).wait_recv()
                f_rdma(c).start()

                la_copy(c, s).wait()
                if c >= 2:
                    od_copy(c - 2, s).wait()
                obd[s] = (la[s] + rx[c].astype(jnp.float32)).astype(
                    jnp.bfloat16
                )
                od_copy(c, s).start()
                if c + 2 < C:
                    la_copy(c + 2, s).start()

            if c >= L:
                d = c - L
                s = d % 2
                f_rdma(d).wait_recv()
                lf_copy(d, s).wait()
                if d >= 2:
                    of_copy(d - 2, s).wait()
                obf[s] = (lf[s] + rf[d].astype(jnp.float32)).astype(
                    jnp.bfloat16
                )
                of_copy(d, s).start()
                if d + 2 < C:
                    lf_copy(d + 2, s).start()

        for c in (C - 2, C - 1):
            od_copy(c, c % 2).wait()
            of_copy(c, c % 2).wait()
        for c in range(C):
            x_rdma(c).wait_send()
            f_rdma(c).wait_send()

    return pl.pallas_call(
        body,
        in_specs=[pl.BlockSpec(memory_space=pl.MemorySpace.ANY)],
        out_specs=pl.BlockSpec(memory_space=pl.MemorySpace.ANY),
        out_shape=jax.ShapeDtypeStruct((M, N), jnp.bfloat16),
        scratch_shapes=[
            pltpu.VMEM((2, R, N), jnp.float32),
            pltpu.VMEM((2, R, N), jnp.float32),
            pltpu.VMEM((2, R, N), jnp.float32),
            pltpu.VMEM((C, R, N), jnp.bfloat16),
            pltpu.VMEM((C, R, N), jnp.bfloat16),
            pltpu.VMEM((C, R, N), jnp.bfloat16),
            pltpu.VMEM((2, R, N), jnp.bfloat16),
            pltpu.VMEM((2, R, N), jnp.bfloat16),
            pltpu.SemaphoreType.DMA((2,)),
            pltpu.SemaphoreType.DMA((2,)),
            pltpu.SemaphoreType.DMA((2,)),
            pltpu.SemaphoreType.DMA((2,)),
            pltpu.SemaphoreType.DMA((2,)),
            pltpu.SemaphoreType.DMA((C,)),
            pltpu.SemaphoreType.DMA((C,)),
            pltpu.SemaphoreType.DMA((C,)),
            pltpu.SemaphoreType.DMA((C,)),
        ],
        compiler_params=pltpu.CompilerParams(collective_id=0),
    )(x)
